# baseline (speedup 1.0000x reference)
import functools

import jax
import jax.numpy as jnp
from jax import lax
from jax.experimental import pallas as pl
from jax.experimental.pallas import tpu as pltpu
from jax.experimental.pallas import tpu_sc as plsc

VOCAB = 1000000
DIM = 64
B = 16384
CTX = 10
NEG = 20
MAX_NORM = 1.0

NC = 2
NS = 16
NW = NC * NS
BPW = B // NW
CHUNK = 32
NCHUNK = BPW // CHUNK
LANES = 16
NV = DIM // LANES
MAX_IDX = 128


def _rsqrt(t):
    i = lax.bitcast_convert_type(t, jnp.int32)
    y = lax.bitcast_convert_type(0x5F3759DF - (i >> 1), jnp.float32)
    for _ in range(3):
        y = y * (1.5 - 0.5 * t * y * y)
    return y


def _sc_body(in_hbm, pos_hbm, neg_hbm, enc_hbm, dec_hbm,
             out_hbm, pos_out_hbm, neg_out_hbm,
             enc_idx, pos_idx, neg_idx, enc_rows, pos_rows, neg_rows,
             out_buf, sem):
    cid = lax.axis_index("c")
    sid = lax.axis_index("s")
    wid = sid * NC + cid

    def chunk_body(i, _):
        eb = wid * BPW + i * CHUNK
        pltpu.sync_copy(in_hbm.at[pl.ds(eb * CTX, CHUNK * CTX)], enc_idx)
        pltpu.sync_copy(pos_hbm.at[pl.ds(eb, CHUNK)], pos_idx)
        pltpu.sync_copy(neg_hbm.at[pl.ds(eb * NEG, CHUNK * NEG)], neg_idx)

        descs = []
        for o in range(0, CHUNK * CTX, MAX_IDX):
            n = min(MAX_IDX, CHUNK * CTX - o)
            descs.append(pltpu.async_copy(
                enc_hbm.at[enc_idx.at[pl.ds(o, n)]],
                enc_rows.at[pl.ds(o, n)], sem))
        descs.append(pltpu.async_copy(
            dec_hbm.at[pos_idx], pos_rows, sem))
        for o in range(0, CHUNK * NEG, MAX_IDX):
            n = min(MAX_IDX, CHUNK * NEG - o)
            descs.append(pltpu.async_copy(
                dec_hbm.at[neg_idx.at[pl.ds(o, n)]],
                neg_rows.at[pl.ds(o, n)], sem))
        for d in descs:
            d.wait()

        def belem(b, _):
            base = b * CTX
            accs = [jnp.zeros((LANES,), jnp.float32) for _ in range(NV)]
            for j in range(CTX):
                r = [enc_rows[base + j, pl.ds(k * LANES, LANES)]
                     for k in range(NV)]
                sq = r[0] * r[0]
                for k in range(1, NV):
                    sq = sq + r[k] * r[k]
                nsq = jnp.sum(sq)
                scale = MAX_NORM * _rsqrt(jnp.maximum(nsq, MAX_NORM * MAX_NORM))
                accs = [accs[k] + scale * r[k] for k in range(NV)]
            for k in range(NV):
                out_buf[b, pl.ds(k * LANES, LANES)] = accs[k] * (1.0 / CTX)
            return 0

        lax.fori_loop(0, CHUNK, belem, 0)

        pltpu.sync_copy(out_buf, out_hbm.at[pl.ds(eb, CHUNK)])
        pltpu.sync_copy(pos_rows, pos_out_hbm.at[pl.ds(eb, CHUNK)])
        pltpu.sync_copy(neg_rows, neg_out_hbm.at[pl.ds(eb * NEG, CHUNK * NEG)])
        return 0

    lax.fori_loop(0, NCHUNK, chunk_body, 0)


_sc_call = pl.kernel(
    _sc_body,
    out_type=(
        jax.ShapeDtypeStruct((B, DIM), jnp.float32),
        jax.ShapeDtypeStruct((B, DIM), jnp.float32),
        jax.ShapeDtypeStruct((B * NEG, DIM), jnp.float32),
    ),
    mesh=plsc.VectorSubcoreMesh(
        core_axis_name="c", subcore_axis_name="s",
        num_cores=NC, num_subcores=NS),
    compiler_params=pltpu.CompilerParams(
        needs_layout_passes=False, use_tc_tiling_on_sc=False),
    scratch_types=(
        pltpu.VMEM((CHUNK * CTX,), jnp.int32),
        pltpu.VMEM((CHUNK,), jnp.int32),
        pltpu.VMEM((CHUNK * NEG,), jnp.int32),
        pltpu.VMEM((CHUNK * CTX, DIM), jnp.float32),
        pltpu.VMEM((CHUNK, DIM), jnp.float32),
        pltpu.VMEM((CHUNK * NEG, DIM), jnp.float32),
        pltpu.VMEM((CHUNK, DIM), jnp.float32),
        pltpu.SemaphoreType.DMA,
    ),
)


@jax.jit
def kernel(input, pos, neg, enc_weight, dec_weight):
    out, pos_out, neg_out = _sc_call(
        input.reshape(-1), pos, neg.reshape(-1), enc_weight, dec_weight)
    return out, pos_out, neg_out.reshape(B, NEG, DIM)


# speedup vs baseline: 1.1284x; 1.1284x over previous
import jax
import jax.numpy as jnp
from jax import lax
from jax.experimental import pallas as pl
from jax.experimental.pallas import tpu as pltpu
from jax.experimental.pallas import tpu_sc as plsc

VOCAB = 1000000
DIM = 64
B = 16384
CTX = 10
NEG = 20
MAX_NORM = 1.0

NC = 2
NS = 16
NW = NC * NS
BPW = B // NW
CHUNK = 32
NCHUNK = BPW // CHUNK
LANES = 16
NV = DIM // LANES
MAX_IDX = 128

NPAIR = VOCAB // 256
TAIL = VOCAB - NPAIR * 256
KP = 8
TBW = 256 * KP
NBLK = NPAIR // KP
XPAIR = NPAIR - NBLK * KP
TSTEPS = NBLK // 2
ROWS_PAD = NPAIR * 128 + TAIL
VOCAB_PAD = 2 * ROWS_PAD


def _pairs(x, n):
    parts = []
    for k in range(n):
        a = x[:, 256 * k:256 * k + 128]
        b = x[:, 256 * k + 128:256 * k + 256]
        parts.append(jnp.concatenate([a.T, b.T], axis=1))
    return jnp.concatenate(parts, axis=0) if len(parts) > 1 else parts[0]


def _tr_body(xt_any, o_any, ia, ib, oa, ob, ix, ox, it, ot,
             sia, sib, soa, sob):
    i = pl.program_id(0)
    a = 2 * i
    b = 2 * i + 1

    def in_cp(blk, buf, sem, w=TBW):
        return pltpu.make_async_copy(
            xt_any.at[:, pl.ds(blk * TBW, w)], buf, sem)

    def out_cp(blk, buf, sem, rows=TBW // 2):
        return pltpu.make_async_copy(
            buf, o_any.at[pl.ds(blk * (TBW // 2), rows), :], sem)

    @pl.when(i == 0)
    def _():
        in_cp(a, ia, sia).start()

    in_cp(b, ib, sib).start()
    in_cp(a, ia, sia).wait()

    @pl.when(i > 0)
    def _():
        out_cp(a - 2, oa, soa).wait()

    oa[...] = _pairs(ia[...], KP)
    out_cp(a, oa, soa).start()

    @pl.when(i + 1 < TSTEPS)
    def _():
        in_cp(a + 2, ia, sia).start()

    in_cp(b, ib, sib).wait()

    @pl.when(i > 0)
    def _():
        out_cp(b - 2, ob, sob).wait()

    ob[...] = _pairs(ib[...], KP)
    out_cp(b, ob, sob).start()

    @pl.when(i == TSTEPS - 1)
    def _():
        in_cp(NBLK, ix, sia, w=256 * XPAIR).start()
        in_cp(NBLK, ix, sia, w=256 * XPAIR).wait()
        ox[...] = _pairs(ix[...], XPAIR)
        out_cp(NBLK, ox, soa, rows=128 * XPAIR).start()
        pltpu.make_async_copy(
            xt_any.at[:, pl.ds(NPAIR * 256, TAIL)], it, sib).start()
        pltpu.make_async_copy(
            xt_any.at[:, pl.ds(NPAIR * 256, TAIL)], it, sib).wait()
        tt = it[...].T
        ot[...] = jnp.concatenate([tt, tt], axis=1)
        pltpu.make_async_copy(
            ot, o_any.at[pl.ds(ROWS_PAD - TAIL, TAIL), :], sob).start()
        out_cp(a, oa, soa).wait()
        out_cp(b, ob, sob).wait()
        out_cp(NBLK, ox, soa, rows=128 * XPAIR).wait()
        pltpu.make_async_copy(
            ot, o_any.at[pl.ds(ROWS_PAD - TAIL, TAIL), :], sob).wait()


_tc_transpose = pl.pallas_call(
    _tr_body,
    grid=(TSTEPS,),
    in_specs=[pl.BlockSpec(memory_space=pl.ANY)],
    out_specs=pl.BlockSpec(memory_space=pl.ANY),
    out_shape=jax.ShapeDtypeStruct((ROWS_PAD, 2 * DIM), jnp.float32),
    scratch_shapes=[
        pltpu.VMEM((DIM, TBW), jnp.float32),
        pltpu.VMEM((DIM, TBW), jnp.float32),
        pltpu.VMEM((TBW // 2, 2 * DIM), jnp.float32),
        pltpu.VMEM((TBW // 2, 2 * DIM), jnp.float32),
        pltpu.VMEM((DIM, 256 * XPAIR), jnp.float32),
        pltpu.VMEM((128 * XPAIR, 2 * DIM), jnp.float32),
        pltpu.VMEM((DIM, TAIL), jnp.float32),
        pltpu.VMEM((TAIL, 2 * DIM), jnp.float32),
        pltpu.SemaphoreType.DMA,
        pltpu.SemaphoreType.DMA,
        pltpu.SemaphoreType.DMA,
        pltpu.SemaphoreType.DMA,
    ],
)


def _rsqrt(t):
    i = lax.bitcast_convert_type(t, jnp.int32)
    y = lax.bitcast_convert_type(0x5F3759DF - (i >> 1), jnp.float32)
    for _ in range(3):
        y = y * (1.5 - 0.5 * t * y * y)
    return y


def _remap(idx_ref, n):
    for o in range(0, n, LANES):
        r = idx_ref[pl.ds(o, LANES)]
        idx_ref[pl.ds(o, LANES)] = (
            (r & -256) + ((r & 127) << 1) + ((r >> 7) & 1))


def _sc_body(in_hbm, pos_hbm, neg_hbm, enc_hbm, dec_hbm,
             out_hbm, pos_out_hbm, neg_out_hbm,
             enc_idx, pos_idx, neg_idx, enc_rows, pos_rows, neg_rows,
             out_buf, sem):
    cid = lax.axis_index("c")
    sid = lax.axis_index("s")
    wid = sid * NC + cid

    def chunk_body(i, _):
        eb = wid * BPW + i * CHUNK
        pltpu.sync_copy(in_hbm.at[pl.ds(eb * CTX, CHUNK * CTX)], enc_idx)
        pltpu.sync_copy(pos_hbm.at[pl.ds(eb, CHUNK)], pos_idx)
        pltpu.sync_copy(neg_hbm.at[pl.ds(eb * NEG, CHUNK * NEG)], neg_idx)
        _remap(enc_idx, CHUNK * CTX)
        _remap(pos_idx, CHUNK)
        _remap(neg_idx, CHUNK * NEG)

        descs = []
        for o in range(0, CHUNK * CTX, MAX_IDX):
            n = min(MAX_IDX, CHUNK * CTX - o)
            descs.append(pltpu.async_copy(
                enc_hbm.at[enc_idx.at[pl.ds(o, n)]],
                enc_rows.at[pl.ds(o, n)], sem))
        descs.append(pltpu.async_copy(
            dec_hbm.at[pos_idx], pos_rows, sem))
        for o in range(0, CHUNK * NEG, MAX_IDX):
            n = min(MAX_IDX, CHUNK * NEG - o)
            descs.append(pltpu.async_copy(
                dec_hbm.at[neg_idx.at[pl.ds(o, n)]],
                neg_rows.at[pl.ds(o, n)], sem))
        for d in descs:
            d.wait()

        def belem(b, _):
            base = b * CTX
            accs = [jnp.zeros((LANES,), jnp.float32) for _ in range(NV)]
            for j in range(CTX):
                r = [enc_rows[base + j, pl.ds(k * LANES, LANES)]
                     for k in range(NV)]
                sq = r[0] * r[0]
                for k in range(1, NV):
                    sq = sq + r[k] * r[k]
                nsq = jnp.sum(sq)
                scale = MAX_NORM * _rsqrt(jnp.maximum(nsq, MAX_NORM * MAX_NORM))
                accs = [accs[k] + scale * r[k] for k in range(NV)]
            for k in range(NV):
                out_buf[b, pl.ds(k * LANES, LANES)] = accs[k] * (1.0 / CTX)
            return 0

        lax.fori_loop(0, CHUNK, belem, 0)

        pltpu.sync_copy(out_buf, out_hbm.at[pl.ds(eb, CHUNK)])
        pltpu.sync_copy(pos_rows, pos_out_hbm.at[pl.ds(eb, CHUNK)])
        pltpu.sync_copy(neg_rows, neg_out_hbm.at[pl.ds(eb * NEG, CHUNK * NEG)])
        return 0

    lax.fori_loop(0, NCHUNK, chunk_body, 0)


_sc_call = pl.kernel(
    _sc_body,
    out_type=(
        jax.ShapeDtypeStruct((B, DIM), jnp.float32),
        jax.ShapeDtypeStruct((B, DIM), jnp.float32),
        jax.ShapeDtypeStruct((B * NEG, DIM), jnp.float32),
    ),
    mesh=plsc.VectorSubcoreMesh(
        core_axis_name="c", subcore_axis_name="s",
        num_cores=NC, num_subcores=NS),
    compiler_params=pltpu.CompilerParams(
        needs_layout_passes=False, use_tc_tiling_on_sc=False),
    scratch_types=(
        pltpu.VMEM((CHUNK * CTX,), jnp.int32),
        pltpu.VMEM((CHUNK,), jnp.int32),
        pltpu.VMEM((CHUNK * NEG,), jnp.int32),
        pltpu.VMEM((CHUNK * CTX, DIM), jnp.float32),
        pltpu.VMEM((CHUNK, DIM), jnp.float32),
        pltpu.VMEM((CHUNK * NEG, DIM), jnp.float32),
        pltpu.VMEM((CHUNK, DIM), jnp.float32),
        pltpu.SemaphoreType.DMA,
    ),
)


@jax.jit
def kernel(input, pos, neg, enc_weight, dec_weight):
    enc_rm = _tc_transpose(enc_weight.T).reshape(VOCAB_PAD, DIM)
    dec_rm = _tc_transpose(dec_weight.T).reshape(VOCAB_PAD, DIM)
    out, pos_out, neg_out = _sc_call(
        input.reshape(-1), pos, neg.reshape(-1), enc_rm, dec_rm)
    return out, pos_out, neg_out.reshape(B, NEG, DIM)


# speedup vs baseline: 1.8564x; 1.6452x over previous
import jax
import jax.numpy as jnp
from jax import lax
from jax.experimental import pallas as pl
from jax.experimental.pallas import tpu as pltpu
from jax.experimental.pallas import tpu_sc as plsc

VOCAB = 1000000
DIM = 64
B = 16384
CTX = 10
NEG = 20
MAX_NORM = 1.0

NC = 2
NS = 16
NW = NC * NS
BPW = B // NW
CHUNK = 16
NCHUNK = BPW // CHUNK
NPAIRS = NCHUNK // 2
LANES = 16
NV = DIM // LANES
MAX_IDX = 128

NPAIR = VOCAB // 256
TAIL = VOCAB - NPAIR * 256
KP = 8
TBW = 256 * KP
NBLK = NPAIR // KP
XPAIR = NPAIR - NBLK * KP
NSLOT = 4
TSTEPS = NBLK // NSLOT
ROWS_PAD = NPAIR * 128 + TAIL
VOCAB_PAD = 2 * ROWS_PAD


def _pairs(x, n):
    parts = []
    for k in range(n):
        a = x[:, 256 * k:256 * k + 128]
        b = x[:, 256 * k + 128:256 * k + 256]
        parts.append(jnp.concatenate([a, b], axis=0).T)
    return jnp.concatenate(parts, axis=0) if len(parts) > 1 else parts[0]


def _tr_body(xt_any, o_any, *refs):
    ibufs = refs[0:NSLOT]
    obufs = refs[NSLOT:2 * NSLOT]
    ix, ot = refs[2 * NSLOT], refs[2 * NSLOT + 1]
    isems = refs[2 * NSLOT + 2:3 * NSLOT + 2]
    osems = refs[3 * NSLOT + 2:4 * NSLOT + 2]
    i = pl.program_id(0)

    def in_cp(blk, buf, sem, w=TBW):
        return pltpu.make_async_copy(
            xt_any.at[:, pl.ds(blk * TBW, w)], buf, sem)

    def out_cp(blk, buf, sem, rows=TBW // 2):
        return pltpu.make_async_copy(
            buf, o_any.at[pl.ds(blk * (TBW // 2), rows), :], sem)

    @pl.when(i == 0)
    def _():
        for j in range(NSLOT):
            in_cp(j, ibufs[j], isems[j]).start()

    for j in range(NSLOT):
        blk = NSLOT * i + j
        in_cp(blk, ibufs[j], isems[j]).wait()

        @pl.when(i > 0)
        def _(j=j, blk=blk):
            out_cp(blk - NSLOT, obufs[j], osems[j]).wait()

        obufs[j][...] = _pairs(ibufs[j][...], KP)
        out_cp(blk, obufs[j], osems[j]).start()

        @pl.when(i + 1 < TSTEPS)
        def _(j=j, blk=blk):
            in_cp(blk + NSLOT, ibufs[j], isems[j]).start()

    @pl.when(i == TSTEPS - 1)
    def _():
        in_cp(NBLK, ix, isems[0], w=256 * XPAIR + TAIL).start()
        in_cp(NBLK, ix, isems[0], w=256 * XPAIR + TAIL).wait()
        ox = _pairs(ix[:, :256 * XPAIR], XPAIR)
        tt = ix[:, 256 * XPAIR:256 * XPAIR + TAIL].T
        ot[...] = jnp.concatenate(
            [ox, jnp.concatenate([tt, tt], axis=1)], axis=0)
        pltpu.make_async_copy(
            ot, o_any.at[pl.ds(NBLK * (TBW // 2), 128 * XPAIR + TAIL), :],
            osems[0]).start()
        for j in range(NSLOT):
            out_cp(NSLOT * i + j, obufs[j], osems[j]).wait()
        pltpu.make_async_copy(
            ot, o_any.at[pl.ds(NBLK * (TBW // 2), 128 * XPAIR + TAIL), :],
            osems[0]).wait()


_tc_transpose = pl.pallas_call(
    _tr_body,
    grid=(TSTEPS,),
    in_specs=[pl.BlockSpec(memory_space=pl.ANY)],
    out_specs=pl.BlockSpec(memory_space=pl.ANY),
    out_shape=jax.ShapeDtypeStruct((ROWS_PAD, 2 * DIM), jnp.float32),
    scratch_shapes=(
        [pltpu.VMEM((DIM, TBW), jnp.float32) for _ in range(NSLOT)]
        + [pltpu.VMEM((TBW // 2, 2 * DIM), jnp.float32)
           for _ in range(NSLOT)]
        + [pltpu.VMEM((DIM, 256 * XPAIR + TAIL), jnp.float32),
           pltpu.VMEM((128 * XPAIR + TAIL, 2 * DIM), jnp.float32)]
        + [pltpu.SemaphoreType.DMA for _ in range(2 * NSLOT)]
    ),
)


def _rsqrt(t):
    i = lax.bitcast_convert_type(t, jnp.int32)
    y = lax.bitcast_convert_type(0x5F3759DF - (i >> 1), jnp.float32)
    for _ in range(3):
        y = y * (1.5 - 0.5 * t * y * y)
    return y


def _remap(idx_ref, n):
    for o in range(0, n, LANES):
        r = idx_ref[pl.ds(o, LANES)]
        idx_ref[pl.ds(o, LANES)] = (
            (r & -256) + ((r & 127) << 1) + ((r >> 7) & 1))


def _sc_body(in_hbm, pos_hbm, neg_hbm, enc_hbm, dec_hbm,
             out_hbm, pos_out_hbm, neg_out_hbm,
             enc_idx_a, pos_idx_a, neg_idx_a, enc_rows_a, pos_rows_a,
             neg_rows_a, out_buf_a,
             enc_idx_b, pos_idx_b, neg_idx_b, enc_rows_b, pos_rows_b,
             neg_rows_b, out_buf_b,
             gsem, ssem):
    cid = lax.axis_index("c")
    sid = lax.axis_index("s")
    wid = sid * NC + cid
    bufs_a = (enc_idx_a, pos_idx_a, neg_idx_a, enc_rows_a, pos_rows_a,
              neg_rows_a, out_buf_a)
    bufs_b = (enc_idx_b, pos_idx_b, neg_idx_b, enc_rows_b, pos_rows_b,
              neg_rows_b, out_buf_b)

    def gather_descs(bufs):
        eidx, pidx, nidx, erows, prows, nrows, _ = bufs
        descs = []
        for o in range(0, CHUNK * CTX, MAX_IDX):
            n = min(MAX_IDX, CHUNK * CTX - o)
            descs.append(pltpu.make_async_copy(
                enc_hbm.at[eidx.at[pl.ds(o, n)]],
                erows.at[pl.ds(o, n)], gsem))
        descs.append(pltpu.make_async_copy(dec_hbm.at[pidx], prows, gsem))
        for o in range(0, CHUNK * NEG, MAX_IDX):
            n = min(MAX_IDX, CHUNK * NEG - o)
            descs.append(pltpu.make_async_copy(
                dec_hbm.at[nidx.at[pl.ds(o, n)]],
                nrows.at[pl.ds(o, n)], gsem))
        return descs

    def stage_and_fire(c, bufs):
        eidx, pidx, nidx, _, _, _, _ = bufs
        eb = wid * BPW + c * CHUNK
        pltpu.sync_copy(in_hbm.at[pl.ds(eb * CTX, CHUNK * CTX)], eidx)
        pltpu.sync_copy(pos_hbm.at[pl.ds(eb, CHUNK)], pidx)
        pltpu.sync_copy(neg_hbm.at[pl.ds(eb * NEG, CHUNK * NEG)], nidx)
        _remap(eidx, CHUNK * CTX)
        _remap(pidx, CHUNK)
        _remap(nidx, CHUNK * NEG)
        for d in gather_descs(bufs):
            d.start()

    def wait_gathers(bufs):
        for d in gather_descs(bufs):
            d.wait()

    def pn_descs(c, bufs):
        _, _, _, _, prows, nrows, _ = bufs
        eb = wid * BPW + c * CHUNK
        return [
            pltpu.make_async_copy(
                prows, pos_out_hbm.at[pl.ds(eb, CHUNK)], ssem),
            pltpu.make_async_copy(
                nrows, neg_out_hbm.at[pl.ds(eb * NEG, CHUNK * NEG)], ssem),
        ]

    def out_desc(c, bufs):
        obuf = bufs[6]
        eb = wid * BPW + c * CHUNK
        return pltpu.make_async_copy(obuf, out_hbm.at[pl.ds(eb, CHUNK)], ssem)

    def wait_stores(c, bufs):
        for d in pn_descs(c, bufs):
            d.wait()
        out_desc(c, bufs).wait()

    def compute(bufs):
        _, _, _, erows, _, _, obuf = bufs

        def belem(b, _):
            base = b * CTX
            accs = [jnp.zeros((LANES,), jnp.float32) for _ in range(NV)]
            for j in range(CTX):
                r = [erows[base + j, pl.ds(k * LANES, LANES)]
                     for k in range(NV)]
                sq = r[0] * r[0]
                for k in range(1, NV):
                    sq = sq + r[k] * r[k]
                nsq = jnp.sum(sq)
                scale = MAX_NORM * _rsqrt(jnp.maximum(nsq, MAX_NORM * MAX_NORM))
                accs = [accs[k] + scale * r[k] for k in range(NV)]
            for k in range(NV):
                obuf[b, pl.ds(k * LANES, LANES)] = accs[k] * (1.0 / CTX)
            return 0

        lax.fori_loop(0, CHUNK, belem, 0)

    stage_and_fire(0, bufs_a)

    def pair_body(i, _):
        a = 2 * i
        b = 2 * i + 1

        wait_gathers(bufs_a)
        for d in pn_descs(a, bufs_a):
            d.start()

        @pl.when(i > 0)
        def _():
            wait_stores(b - 2, bufs_b)

        stage_and_fire(b, bufs_b)
        compute(bufs_a)
        out_desc(a, bufs_a).start()

        wait_gathers(bufs_b)
        for d in pn_descs(b, bufs_b):
            d.start()

        @pl.when(i + 1 < NPAIRS)
        def _():
            wait_stores(a, bufs_a)
            stage_and_fire(a + 2, bufs_a)

        compute(bufs_b)
        out_desc(b, bufs_b).start()
        return 0

    lax.fori_loop(0, NPAIRS, pair_body, 0)
    wait_stores(NCHUNK - 2, bufs_a)
    wait_stores(NCHUNK - 1, bufs_b)


_sc_call = pl.kernel(
    _sc_body,
    out_type=(
        jax.ShapeDtypeStruct((B, DIM), jnp.float32),
        jax.ShapeDtypeStruct((B, DIM), jnp.float32),
        jax.ShapeDtypeStruct((B * NEG, DIM), jnp.float32),
    ),
    mesh=plsc.VectorSubcoreMesh(
        core_axis_name="c", subcore_axis_name="s",
        num_cores=NC, num_subcores=NS),
    compiler_params=pltpu.CompilerParams(
        needs_layout_passes=False, use_tc_tiling_on_sc=False),
    scratch_types=(
        pltpu.VMEM((CHUNK * CTX,), jnp.int32),
        pltpu.VMEM((CHUNK,), jnp.int32),
        pltpu.VMEM((CHUNK * NEG,), jnp.int32),
        pltpu.VMEM((CHUNK * CTX, DIM), jnp.float32),
        pltpu.VMEM((CHUNK, DIM), jnp.float32),
        pltpu.VMEM((CHUNK * NEG, DIM), jnp.float32),
        pltpu.VMEM((CHUNK, DIM), jnp.float32),
        pltpu.VMEM((CHUNK * CTX,), jnp.int32),
        pltpu.VMEM((CHUNK,), jnp.int32),
        pltpu.VMEM((CHUNK * NEG,), jnp.int32),
        pltpu.VMEM((CHUNK * CTX, DIM), jnp.float32),
        pltpu.VMEM((CHUNK, DIM), jnp.float32),
        pltpu.VMEM((CHUNK * NEG, DIM), jnp.float32),
        pltpu.VMEM((CHUNK, DIM), jnp.float32),
        pltpu.SemaphoreType.DMA,
        pltpu.SemaphoreType.DMA,
    ),
)


@jax.jit
def kernel(input, pos, neg, enc_weight, dec_weight):
    enc_rm = _tc_transpose(enc_weight.T).reshape(VOCAB_PAD, DIM)
    dec_rm = _tc_transpose(dec_weight.T).reshape(VOCAB_PAD, DIM)
    out, pos_out, neg_out = _sc_call(
        input.reshape(-1), pos, neg.reshape(-1), enc_rm, dec_rm)
    return out, pos_out, neg_out.reshape(B, NEG, DIM)


# speedup vs baseline: 2.4816x; 1.3367x over previous
import jax
import jax.numpy as jnp
from jax import lax
from jax.experimental import pallas as pl
from jax.experimental.pallas import tpu as pltpu
from jax.experimental.pallas import tpu_sc as plsc

VOCAB = 1000000
DIM = 64
B = 16384
CTX = 10
NEG = 20
MAX_NORM = 1.0

NC = 2
NS = 16
NW = NC * NS
BPW = B // NW
CHUNK = 16
NCHUNK = BPW // CHUNK
NPAIRS = NCHUNK // 2
LANES = 16
NV = DIM // LANES
MAX_IDX = 128

NPAIR = VOCAB // 256
TAIL = VOCAB - NPAIR * 256
KP = 8
TBW = 256 * KP
NBLK = NPAIR // KP
XPAIR = NPAIR - NBLK * KP
NSLOT = 8
TSTEPS = NBLK // NSLOT
ROWS_PAD = NPAIR * 128 + TAIL
VOCAB_PAD = 2 * ROWS_PAD


def _pairs(x, n):
    parts = []
    for k in range(n):
        a = x[:, 256 * k:256 * k + 128]
        b = x[:, 256 * k + 128:256 * k + 256]
        parts.append(jnp.concatenate([a, b], axis=0).T)
    return jnp.concatenate(parts, axis=0) if len(parts) > 1 else parts[0]


def _tr_body(xt_any, o_any, *refs):
    ibufs = refs[0:NSLOT]
    obufs = refs[NSLOT:2 * NSLOT]
    ix, ot = refs[2 * NSLOT], refs[2 * NSLOT + 1]
    isems = refs[2 * NSLOT + 2:3 * NSLOT + 2]
    osems = refs[3 * NSLOT + 2:4 * NSLOT + 2]
    i = pl.program_id(0)

    def in_cp(blk, buf, sem, w=TBW):
        return pltpu.make_async_copy(
            xt_any.at[:, pl.ds(blk * TBW, w)], buf, sem)

    def out_cp(blk, buf, sem, rows=TBW // 2):
        return pltpu.make_async_copy(
            buf, o_any.at[pl.ds(blk * (TBW // 2), rows), :], sem)

    @pl.when(i == 0)
    def _():
        for j in range(NSLOT):
            in_cp(j, ibufs[j], isems[j]).start()

    for j in range(NSLOT):
        blk = NSLOT * i + j
        in_cp(blk, ibufs[j], isems[j]).wait()

        @pl.when(i > 0)
        def _(j=j, blk=blk):
            out_cp(blk - NSLOT, obufs[j], osems[j]).wait()

        obufs[j][...] = _pairs(ibufs[j][...], KP)
        out_cp(blk, obufs[j], osems[j]).start()

        @pl.when(i + 1 < TSTEPS)
        def _(j=j, blk=blk):
            in_cp(blk + NSLOT, ibufs[j], isems[j]).start()

    @pl.when(i == TSTEPS - 1)
    def _():
        in_cp(NBLK, ix, isems[0], w=256 * XPAIR + TAIL).start()
        in_cp(NBLK, ix, isems[0], w=256 * XPAIR + TAIL).wait()
        ox = _pairs(ix[:, :256 * XPAIR], XPAIR)
        tt = ix[:, 256 * XPAIR:256 * XPAIR + TAIL].T
        ot[...] = jnp.concatenate(
            [ox, jnp.concatenate([tt, tt], axis=1)], axis=0)
        pltpu.make_async_copy(
            ot, o_any.at[pl.ds(NBLK * (TBW // 2), 128 * XPAIR + TAIL), :],
            osems[0]).start()
        for j in range(NSLOT):
            out_cp(NSLOT * i + j, obufs[j], osems[j]).wait()
        pltpu.make_async_copy(
            ot, o_any.at[pl.ds(NBLK * (TBW // 2), 128 * XPAIR + TAIL), :],
            osems[0]).wait()


_tc_transpose = pl.pallas_call(
    _tr_body,
    grid=(TSTEPS,),
    in_specs=[pl.BlockSpec(memory_space=pl.ANY)],
    out_specs=pl.BlockSpec(memory_space=pl.ANY),
    out_shape=jax.ShapeDtypeStruct((ROWS_PAD, 2 * DIM), jnp.float32),
    scratch_shapes=(
        [pltpu.VMEM((DIM, TBW), jnp.float32) for _ in range(NSLOT)]
        + [pltpu.VMEM((TBW // 2, 2 * DIM), jnp.float32)
           for _ in range(NSLOT)]
        + [pltpu.VMEM((DIM, 256 * XPAIR + TAIL), jnp.float32),
           pltpu.VMEM((128 * XPAIR + TAIL, 2 * DIM), jnp.float32)]
        + [pltpu.SemaphoreType.DMA for _ in range(2 * NSLOT)]
    ),
)


NW_BLK = 2048


def _negt_body(x_ref, o_ref):
    o_ref[0] = x_ref[0].T


_neg_post = pl.pallas_call(
    _negt_body,
    grid=(NEG // 2, B // NW_BLK),
    in_specs=[pl.BlockSpec((1, NW_BLK, 2 * DIM), lambda j, i: (j, i, 0))],
    out_specs=pl.BlockSpec((1, 2 * DIM, NW_BLK), lambda j, i: (j, 0, i)),
    out_shape=jax.ShapeDtypeStruct((NEG // 2, 2 * DIM, B), jnp.float32),
)


def _rsqrt(t):
    i = lax.bitcast_convert_type(t, jnp.int32)
    y = lax.bitcast_convert_type(0x5F3759DF - (i >> 1), jnp.float32)
    for _ in range(3):
        y = y * (1.5 - 0.5 * t * y * y)
    return y


def _table_remap(r):
    return (r & -256) + ((r & 127) << 1) + ((r >> 7) & 1)


def _remap(idx_ref, n):
    for o in range(0, n, LANES):
        idx_ref[pl.ds(o, LANES)] = _table_remap(idx_ref[pl.ds(o, LANES)])


def _sc_body(in_hbm, pos_hbm, neg_hbm, enc_hbm, dec_hbm,
             out_hbm, pos_out_hbm, neg_out_hbm,
             enc_idx_a, pos_idx_a, neg_idx_a, enc_rows_a, pos_rows_a,
             neg_rows_a, out_buf_a, nstage_a,
             enc_idx_b, pos_idx_b, neg_idx_b, enc_rows_b, pos_rows_b,
             neg_rows_b, out_buf_b, nstage_b,
             gsem, ssem):
    cid = lax.axis_index("c")
    sid = lax.axis_index("s")
    wid = sid * NC + cid
    bufs_a = (enc_idx_a, pos_idx_a, neg_idx_a, enc_rows_a, pos_rows_a,
              neg_rows_a, out_buf_a, nstage_a)
    bufs_b = (enc_idx_b, pos_idx_b, neg_idx_b, enc_rows_b, pos_rows_b,
              neg_rows_b, out_buf_b, nstage_b)

    def gather_descs(bufs):
        eidx, pidx, nidx, erows, prows, nrows = bufs[:6]
        descs = []
        for o in range(0, CHUNK * CTX, MAX_IDX):
            n = min(MAX_IDX, CHUNK * CTX - o)
            descs.append(pltpu.make_async_copy(
                enc_hbm.at[eidx.at[pl.ds(o, n)]],
                erows.at[pl.ds(o, n)], gsem))
        descs.append(pltpu.make_async_copy(dec_hbm.at[pidx], prows, gsem))
        for o in range(0, CHUNK * NEG, MAX_IDX):
            n = min(MAX_IDX, CHUNK * NEG - o)
            descs.append(pltpu.make_async_copy(
                dec_hbm.at[nidx.at[pl.ds(o, n)]],
                nrows.at[pl.ds(o, n)], gsem))
        return descs

    def stage_and_fire(c, bufs):
        eidx, pidx, nidx = bufs[0], bufs[1], bufs[2]
        nstage = bufs[7]
        eb = wid * BPW + c * CHUNK
        pltpu.sync_copy(in_hbm.at[pl.ds(eb * CTX, CHUNK * CTX)], eidx)
        pltpu.sync_copy(pos_hbm.at[pl.ds(eb, CHUNK)], pidx)
        pltpu.sync_copy(neg_hbm.at[:, pl.ds(eb, CHUNK)], nstage)
        _remap(eidx, CHUNK * CTX)
        _remap(pidx, CHUNK)
        lane = lax.iota(jnp.int32, LANES)
        sh = (2 * CHUNK).bit_length() - 1
        for s in range(CHUNK * NEG // LANES):
            p = s * LANES + lane
            jp = p >> sh
            rem = p & (2 * CHUNK - 1)
            bb = rem >> 1
            half = rem & 1
            r = plsc.load_gather(nstage, [2 * jp + half, bb])
            nidx[pl.ds(s * LANES, LANES)] = _table_remap(r)
        for d in gather_descs(bufs):
            d.start()

    def wait_gathers(bufs):
        for d in gather_descs(bufs):
            d.wait()

    def pn_descs(c, bufs):
        prows, nrows = bufs[4], bufs[5]
        eb = wid * BPW + c * CHUNK
        descs = [pltpu.make_async_copy(
            prows, pos_out_hbm.at[pl.ds(eb, CHUNK)], ssem)]
        for jp in range(NEG // 2):
            descs.append(pltpu.make_async_copy(
                nrows.at[pl.ds(jp * 2 * CHUNK, 2 * CHUNK)],
                neg_out_hbm.at[pl.ds(jp * 2 * B + 2 * eb, 2 * CHUNK)],
                ssem))
        return descs

    def out_desc(c, bufs):
        obuf = bufs[6]
        eb = wid * BPW + c * CHUNK
        return pltpu.make_async_copy(obuf, out_hbm.at[pl.ds(eb, CHUNK)], ssem)

    def wait_stores(c, bufs):
        for d in pn_descs(c, bufs):
            d.wait()
        out_desc(c, bufs).wait()

    def compute(bufs):
        erows, obuf = bufs[3], bufs[6]

        def belem(b, _):
            base = b * CTX
            accs = [jnp.zeros((LANES,), jnp.float32) for _ in range(NV)]
            for j in range(CTX):
                r = [erows[base + j, pl.ds(k * LANES, LANES)]
                     for k in range(NV)]
                sq = r[0] * r[0]
                for k in range(1, NV):
                    sq = sq + r[k] * r[k]
                nsq = jnp.sum(sq)
                scale = MAX_NORM * _rsqrt(jnp.maximum(nsq, MAX_NORM * MAX_NORM))
                accs = [accs[k] + scale * r[k] for k in range(NV)]
            for k in range(NV):
                obuf[b, pl.ds(k * LANES, LANES)] = accs[k] * (1.0 / CTX)
            return 0

        lax.fori_loop(0, CHUNK, belem, 0)

    stage_and_fire(0, bufs_a)

    def pair_body(i, _):
        a = 2 * i
        b = 2 * i + 1

        wait_gathers(bufs_a)
        for d in pn_descs(a, bufs_a):
            d.start()

        @pl.when(i > 0)
        def _():
            wait_stores(b - 2, bufs_b)

        stage_and_fire(b, bufs_b)
        compute(bufs_a)
        out_desc(a, bufs_a).start()

        wait_gathers(bufs_b)
        for d in pn_descs(b, bufs_b):
            d.start()

        @pl.when(i + 1 < NPAIRS)
        def _():
            wait_stores(a, bufs_a)
            stage_and_fire(a + 2, bufs_a)

        compute(bufs_b)
        out_desc(b, bufs_b).start()
        return 0

    lax.fori_loop(0, NPAIRS, pair_body, 0)
    wait_stores(NCHUNK - 2, bufs_a)
    wait_stores(NCHUNK - 1, bufs_b)


_sc_call = pl.kernel(
    _sc_body,
    out_type=(
        jax.ShapeDtypeStruct((B, DIM), jnp.float32),
        jax.ShapeDtypeStruct((B, DIM), jnp.float32),
        jax.ShapeDtypeStruct((B * NEG, DIM), jnp.float32),
    ),
    mesh=plsc.VectorSubcoreMesh(
        core_axis_name="c", subcore_axis_name="s",
        num_cores=NC, num_subcores=NS),
    compiler_params=pltpu.CompilerParams(
        needs_layout_passes=False, use_tc_tiling_on_sc=False),
    scratch_types=(
        pltpu.VMEM((CHUNK * CTX,), jnp.int32),
        pltpu.VMEM((CHUNK,), jnp.int32),
        pltpu.VMEM((CHUNK * NEG,), jnp.int32),
        pltpu.VMEM((CHUNK * CTX, DIM), jnp.float32),
        pltpu.VMEM((CHUNK, DIM), jnp.float32),
        pltpu.VMEM((CHUNK * NEG, DIM), jnp.float32),
        pltpu.VMEM((CHUNK, DIM), jnp.float32),
        pltpu.VMEM((NEG, CHUNK), jnp.int32),
        pltpu.VMEM((CHUNK * CTX,), jnp.int32),
        pltpu.VMEM((CHUNK,), jnp.int32),
        pltpu.VMEM((CHUNK * NEG,), jnp.int32),
        pltpu.VMEM((CHUNK * CTX, DIM), jnp.float32),
        pltpu.VMEM((CHUNK, DIM), jnp.float32),
        pltpu.VMEM((CHUNK * NEG, DIM), jnp.float32),
        pltpu.VMEM((CHUNK, DIM), jnp.float32),
        pltpu.VMEM((NEG, CHUNK), jnp.int32),
        pltpu.SemaphoreType.DMA,
        pltpu.SemaphoreType.DMA,
    ),
)


@jax.jit
def kernel(input, pos, neg, enc_weight, dec_weight):
    enc_rm = _tc_transpose(enc_weight.T).reshape(VOCAB_PAD, DIM)
    dec_rm = _tc_transpose(dec_weight.T).reshape(VOCAB_PAD, DIM)
    out, pos_out, neg_jp = _sc_call(
        input.reshape(-1), pos, neg.T, enc_rm, dec_rm)
    neg_t = _neg_post(neg_jp.reshape(NEG // 2, B, 2 * DIM))
    neg_out = neg_t.reshape(NEG, DIM, B).transpose(2, 0, 1)
    return out, pos_out, neg_out
